# baseline (speedup 1.0000x reference)
import jax
import jax.numpy as jnp
from jax.experimental import pallas as pl

N_FLOW = 10000
N_MEMB = 2000
EMB = 16
ATTN = 16
TDIM = 16
E_FF = 160000


def _timestep_embedding(tau, dim):
    half = dim // 2
    freqs = jnp.exp(-jnp.log(10000.0) * jnp.arange(half) / half)
    args = tau[:, None] * freqs[None, :]
    return jnp.concatenate([jnp.sin(args), jnp.cos(args)], axis=1)


def _dec_kernel(y_ref, w_ref, b_ref, o_ref):
    o_ref[...] = y_ref[...] @ w_ref[...] + b_ref[...]


def _pallas_dec(y, W, b):
    n, fi = y.shape
    fo = W.shape[1]
    return pl.pallas_call(
        _dec_kernel,
        out_shape=jax.ShapeDtypeStruct((n, fo), jnp.float32),
    )(y, W, b[None, :])


def kernel(flow_x, memb_y, tau, edge_index_f2f, edge_attr_f2f, edge_index_m2f, edge_attr_m2f, weights):
    w = weights
    temb = _timestep_embedding(tau, TDIM)
    temb_flow = jnp.repeat(temb, N_FLOW, axis=0)
    temb_memb = jnp.repeat(temb, N_MEMB, axis=0)
    y_memb = jnp.concatenate([memb_y, temb_memb], axis=1) @ w['enc_memb_W1'] + w['enc_memb_b1']
    y_memb = jax.nn.relu(y_memb)
    y_memb = jax.nn.relu(y_memb @ w['enc_memb_W2'] + w['enc_memb_b2'])
    x_flow = jnp.concatenate([flow_x, temb_flow], axis=1) @ w['enc_flow_W'] + w['enc_flow_b']
    tau_nodes = jnp.repeat(tau.reshape(-1, 1), N_FLOW, axis=0)
    src_ff, dst_ff = edge_index_f2f[0], edge_index_f2f[1]
    src_mf, dst_mf = edge_index_m2f[0], edge_index_m2f[1]
    ea_ff = edge_attr_f2f
    ea_mf = edge_attr_m2f
    y_flow = x_flow
    for l in w['layers']:
        q = x_flow @ l['Wq']
        kk = y_memb @ l['Wk']
        v = y_memb @ l['Wv']
        e = jnp.sum(q[dst_mf] * kk[src_mf], axis=1) / jnp.sqrt(float(ATTN)) + (ea_mf @ l['We'])[:, 0]
        emax = jax.ops.segment_max(e, dst_mf, num_segments=N_FLOW)
        emax = jnp.where(jnp.isfinite(emax), emax, 0.0)
        a = jnp.exp(e - emax[dst_mf])
        denom = jax.ops.segment_sum(a, dst_mf, num_segments=N_FLOW)
        msg_c = jax.ops.segment_sum(a[:, None] * v[src_mf], dst_mf, num_segments=N_FLOW)
        x_cross = (msg_c / jnp.clip(denom, 1e-9)[:, None]) @ l['Wo']
        h = jax.nn.relu(ea_ff @ l['kW1'] + l['kb1'])
        h = jax.nn.relu(h @ l['kW2'] + l['kb2'])
        K = (h @ l['kW3'] + l['kb3']).reshape(-1, EMB, EMB)
        m = jnp.einsum('eij,ej->ei', K, x_flow[src_ff])
        deg = jax.ops.segment_sum(jnp.ones(E_FF, jnp.float32), dst_ff, num_segments=N_FLOW)
        aggr = jax.ops.segment_sum(m, dst_ff, num_segments=N_FLOW) / jnp.clip(deg, 1.0)[:, None]
        x_intra = x_flow @ l['Wroot'] + l['broot'] + aggr
        gno = jax.nn.relu(x_intra + x_cross)
        tc = jax.nn.silu(tau_nodes @ l['tcW1'] + l['tcb1']) @ l['tcW2'] + l['tcb2']
        scale, shift = tc[:, :EMB], tc[:, EMB:]
        y_flow = jax.nn.silu(gno * (1.0 + scale) + shift)
    y_memb_out = y_memb @ w['dec_memb_W'] + w['dec_memb_b']
    y_flow_out = _pallas_dec(y_flow + x_flow, w['dec_flow_W'], w['dec_flow_b'])
    return (y_flow_out, y_memb_out)


# speedup vs baseline: 2.1414x; 2.1414x over previous
import functools

import jax
import jax.numpy as jnp
from jax import lax
from jax.experimental import pallas as pl
from jax.experimental.pallas import tpu as pltpu
from jax.experimental.pallas import tpu_sc as plsc

F32 = jnp.float32
N_FLOW = 10000
N_MEMB = 2000
E_FF = 160000
E_MF = 80000
EMB = 16
ATTN = 16
TDIM = 16

NW = 32
CHUNK = 128
FF_PER_W = 5120
MF_PER_W = 2560
E_FF_P = NW * FF_PER_W
E_MF_P = NW * MF_PER_W
FF_CHUNKS = FF_PER_W // CHUNK
MF_CHUNKS = MF_PER_W // CHUNK
NT = 10112
NT_PER_S = NT // 16
DUMMY = 10008

BE = 2048
BF = 2048
BN = 2000

@functools.cache
def _sc_mesh():
    return plsc.VectorSubcoreMesh(core_axis_name="c", subcore_axis_name="s",
                                  num_cores=2, num_subcores=16)


def _enc_body(flow_x, memb_y, cf_row, cm_row, wf, w1m, w2m, b2, wq, wdm, bdm,
              x_flow, tab_a, tab_b, ym_out):
    xf = jnp.dot(flow_x[...], wf[...], preferred_element_type=F32) + cf_row[...]
    x_flow[...] = xf
    tab_a[:, :EMB] = xf
    tab_a[:, EMB:2 * EMB] = jnp.dot(xf, wq[...], preferred_element_type=F32)
    tab_a[:, 2 * EMB:] = jnp.zeros((N_FLOW, 128 - 2 * EMB), F32)
    ym1 = jax.nn.relu(jnp.dot(memb_y[...], w1m[...], preferred_element_type=F32) + cm_row[...])
    ym2 = jax.nn.relu(jnp.dot(ym1, w2m[...], preferred_element_type=F32) + b2[...])
    tab_b[:, :EMB] = ym2
    tab_b[:, EMB:] = jnp.zeros((N_MEMB, 128 - EMB), F32)
    ym_out[...] = jnp.dot(ym2, wdm[...], preferred_element_type=F32) + bdm[...]


def _enc_call(flow_x, memb_y, cf_row, cm_row, w):
    wf = w["enc_flow_W"][:4]
    w1m = w["enc_memb_W1"][:3]
    w2m = w["enc_memb_W2"]
    b2 = w["enc_memb_b2"][None, :]
    wq = w["layers"][-1]["Wq"]
    wdm = w["dec_memb_W"]
    bdm = w["dec_memb_b"][None, :]
    return pl.pallas_call(
        _enc_body,
        out_shape=(
            jax.ShapeDtypeStruct((N_FLOW, EMB), F32),
            jax.ShapeDtypeStruct((N_FLOW, 128), F32),
            jax.ShapeDtypeStruct((N_MEMB, 128), F32),
            jax.ShapeDtypeStruct((N_MEMB, 3), F32),
        ),
    )(flow_x, memb_y, cf_row, cm_row, wf, w1m, w2m, b2, wq, wdm, bdm)


def _gather_body(taba_hbm, tabb_hbm, src3, dst3, srcm3,
                 xs_out, qd_out, ys_out,
                 idx_ff, idx_mf1, idx_mf2, rows_v, sem):
    cid = lax.axis_index("c")
    sid = lax.axis_index("s")
    wid = cid * 16 + sid

    pltpu.sync_copy(src3.at[wid], idx_ff)
    pltpu.sync_copy(dst3.at[wid], idx_mf1)
    pltpu.sync_copy(srcm3.at[wid], idx_mf2)

    ff_base = wid * FF_PER_W
    mf_base = wid * MF_PER_W

    def ff_step(c, _):
        pltpu.async_copy(taba_hbm.at[idx_ff.at[c]], rows_v, sem).wait()
        pltpu.sync_copy(rows_v, xs_out.at[pl.ds(ff_base + c * CHUNK, CHUNK)])
        return _

    lax.fori_loop(0, FF_CHUNKS, ff_step, 0)

    def qd_step(c, _):
        pltpu.async_copy(taba_hbm.at[idx_mf1.at[c]], rows_v, sem).wait()
        pltpu.sync_copy(rows_v, qd_out.at[pl.ds(mf_base + c * CHUNK, CHUNK)])
        return _

    lax.fori_loop(0, MF_CHUNKS, qd_step, 0)

    def ys_step(c, _):
        pltpu.async_copy(tabb_hbm.at[idx_mf2.at[c]], rows_v, sem).wait()
        pltpu.sync_copy(rows_v, ys_out.at[pl.ds(mf_base + c * CHUNK, CHUNK)])
        return _

    lax.fori_loop(0, MF_CHUNKS, ys_step, 0)


def _gather_call(tab_a, tab_b, src3, dst3, srcm3):
    call = pl.kernel(
        _gather_body,
        out_type=(
            jax.ShapeDtypeStruct((E_FF_P, 128), F32),
            jax.ShapeDtypeStruct((E_MF_P, 128), F32),
            jax.ShapeDtypeStruct((E_MF_P, 128), F32),
        ),
        mesh=_sc_mesh(),
        scratch_types=[
            pltpu.VMEM((FF_CHUNKS, CHUNK), jnp.int32),
            pltpu.VMEM((MF_CHUNKS, CHUNK), jnp.int32),
            pltpu.VMEM((MF_CHUNKS, CHUNK), jnp.int32),
            pltpu.VMEM((CHUNK, 128), F32),
            pltpu.SemaphoreType.DMA,
        ],
    )
    return call(tab_a, tab_b, src3, dst3, srcm3)


def _att_e_body(qd, ys, ea, wk, wv, we, vs_out, e_out, gmax_out):
    i = pl.program_id(0)
    ysb = ys[:, :EMB]
    qdb = qd[:, EMB:2 * EMB]
    ks = jnp.dot(ysb, wk[...], preferred_element_type=F32)
    vs = jnp.dot(ysb, wv[...], preferred_element_type=F32)
    vs_out[...] = vs
    ew = jnp.dot(ea[...], we[...], preferred_element_type=F32)
    e = jnp.sum(qdb * ks, axis=1) * (1.0 / 4.0) + ew[:, 0]
    e_out[0, 0, :] = e
    bmax = jnp.max(e)

    @pl.when(i == 0)
    def _():
        gmax_out[...] = jnp.full((1, 1), bmax, F32)

    @pl.when(i > 0)
    def _():
        gmax_out[...] = jnp.maximum(gmax_out[...], bmax)


def _att_e_call(qd, ys, ea_mf_p, l):
    grid = E_MF_P // BE
    return pl.pallas_call(
        _att_e_body,
        grid=(grid,),
        in_specs=[
            pl.BlockSpec((BE, 128), lambda i: (i, 0)),
            pl.BlockSpec((BE, 128), lambda i: (i, 0)),
            pl.BlockSpec((BE, 4), lambda i: (i, 0)),
            pl.BlockSpec((ATTN, ATTN), lambda i: (0, 0)),
            pl.BlockSpec((EMB, EMB), lambda i: (0, 0)),
            pl.BlockSpec((4, 1), lambda i: (0, 0)),
        ],
        out_specs=[
            pl.BlockSpec((BE, EMB), lambda i: (i, 0)),
            pl.BlockSpec((1, 1, BE), lambda i: (i, 0, 0)),
            pl.BlockSpec((1, 1), lambda i: (0, 0)),
        ],
        out_shape=[
            jax.ShapeDtypeStruct((E_MF_P, EMB), F32),
            jax.ShapeDtypeStruct((grid, 1, BE), F32),
            jax.ShapeDtypeStruct((1, 1), F32),
        ],
    )(qd, ys, ea_mf_p, l["Wk"], l["Wv"], l["We"])


def _att_a_body(e3, vs, gmax, p_out):
    a = jnp.exp(e3[0, 0, :] - gmax[0, 0])
    p_out[:, :EMB] = a[:, None] * vs[...]
    p_out[:, EMB:EMB + 1] = a[:, None]
    p_out[:, EMB + 1:] = jnp.zeros((BE, 128 - EMB - 1), F32)


def _att_a_call(e3, vs, gmax):
    grid = E_MF_P // BE
    return pl.pallas_call(
        _att_a_body,
        grid=(grid,),
        in_specs=[
            pl.BlockSpec((1, 1, BE), lambda i: (i, 0, 0)),
            pl.BlockSpec((BE, EMB), lambda i: (i, 0)),
            pl.BlockSpec((1, 1), lambda i: (0, 0)),
        ],
        out_specs=[pl.BlockSpec((BE, 128), lambda i: (i, 0))],
        out_shape=[jax.ShapeDtypeStruct((E_MF_P, 128), F32)],
    )(e3, vs, gmax)[0]


def _gno_body(ea, xs, w1, b1, w2, b2, w3, b3, p_out):
    h = jax.nn.relu(jnp.dot(ea[...], w1[...], preferred_element_type=F32) + b1[...])
    h = jax.nn.relu(jnp.dot(h, w2[...], preferred_element_type=F32) + b2[...])
    k = jnp.dot(h, w3[...], preferred_element_type=F32) + b3[...]
    xt = jnp.concatenate([xs[:, :EMB]] * EMB, axis=1)
    grp = lax.broadcasted_iota(jnp.int32, (EMB * EMB, EMB), 0) // EMB
    col = lax.broadcasted_iota(jnp.int32, (EMB * EMB, EMB), 1)
    s = (grp == col).astype(F32)
    m = jnp.dot(k * xt, s, preferred_element_type=F32)
    p_out[:, :EMB] = m
    p_out[:, EMB:EMB + 1] = jnp.ones((BF, 1), F32)
    p_out[:, EMB + 1:] = jnp.zeros((BF, 128 - EMB - 1), F32)


def _gno_call(ea_ff_p, xs, l):
    grid = E_FF_P // BF
    return pl.pallas_call(
        _gno_body,
        grid=(grid,),
        in_specs=[
            pl.BlockSpec((BF, 4), lambda i: (i, 0)),
            pl.BlockSpec((BF, 128), lambda i: (i, 0)),
            pl.BlockSpec((4, 128), lambda i: (0, 0)),
            pl.BlockSpec((1, 128), lambda i: (0, 0)),
            pl.BlockSpec((128, 128), lambda i: (0, 0)),
            pl.BlockSpec((1, 128), lambda i: (0, 0)),
            pl.BlockSpec((128, 256), lambda i: (0, 0)),
            pl.BlockSpec((1, 256), lambda i: (0, 0)),
        ],
        out_specs=[pl.BlockSpec((BF, 128), lambda i: (i, 0))],
        out_shape=[jax.ShapeDtypeStruct((E_FF_P, 128), F32)],
    )(ea_ff_p, xs, l["kW1"], l["kb1"][None, :], l["kW2"], l["kb2"][None, :],
      l["kW3"], l["kb3"][None, :])[0]


def _scatter_body(pff_hbm, pmf_hbm, dstf3, dstm3, zeros_hbm,
                  tabff_out, tabmf_out,
                  idx_ff, idx_mf, pbuf, tab_sh, sem):
    cid = lax.axis_index("c")
    sid = lax.axis_index("s")
    wid = cid * 16 + sid
    zrow = pl.ds(sid * NT_PER_S, NT_PER_S)

    pltpu.sync_copy(dstf3.at[wid], idx_ff)
    pltpu.sync_copy(dstm3.at[wid], idx_mf)

    pltpu.sync_copy(zeros_hbm.at[zrow], tab_sh.at[zrow])
    plsc.subcore_barrier()
    ff_base = wid * FF_PER_W

    def ff_step(c, _):
        pltpu.sync_copy(pff_hbm.at[pl.ds(ff_base + c * CHUNK, CHUNK)], pbuf)
        pltpu.sync_copy(pbuf, tab_sh.at[idx_ff.at[c]], add=True)
        return _

    lax.fori_loop(0, FF_CHUNKS, ff_step, 0)
    plsc.subcore_barrier()
    pltpu.sync_copy(tab_sh.at[zrow], tabff_out.at[cid, zrow])
    plsc.subcore_barrier()

    pltpu.sync_copy(zeros_hbm.at[zrow], tab_sh.at[zrow])
    plsc.subcore_barrier()
    mf_base = wid * MF_PER_W

    def mf_step(c, _):
        pltpu.sync_copy(pmf_hbm.at[pl.ds(mf_base + c * CHUNK, CHUNK)], pbuf)
        pltpu.sync_copy(pbuf, tab_sh.at[idx_mf.at[c]], add=True)
        return _

    lax.fori_loop(0, MF_CHUNKS, mf_step, 0)
    plsc.subcore_barrier()
    pltpu.sync_copy(tab_sh.at[zrow], tabmf_out.at[cid, zrow])


def _scatter_call(pff, pmf, dstf3, dstm3, zeros_hbm):
    call = pl.kernel(
        _scatter_body,
        out_type=(
            jax.ShapeDtypeStruct((2, NT, 128), F32),
            jax.ShapeDtypeStruct((2, NT, 128), F32),
        ),
        mesh=_sc_mesh(),
        scratch_types=[
            pltpu.VMEM((FF_CHUNKS, CHUNK), jnp.int32),
            pltpu.VMEM((MF_CHUNKS, CHUNK), jnp.int32),
            pltpu.VMEM((CHUNK, 128), F32),
            pltpu.VMEM_SHARED((NT, 128), F32),
            pltpu.SemaphoreType.DMA,
        ],
    )
    return call(pff, pmf, dstf3, dstm3, zeros_hbm)


def _combine_body(x, tabff, tabmf, wroot, broot, wo, scale, shift, wdec, bdec, y_out):
    tf = tabff[0] + tabff[1]
    tm = tabmf[0] + tabmf[1]
    aggr = tf[:, :EMB]
    deg = jnp.maximum(tf[:, EMB:EMB + 1], 1.0)
    msg = tm[:, :EMB]
    denom = jnp.maximum(tm[:, EMB:EMB + 1], 1e-30)
    x_cross = jnp.dot(msg / denom, wo[...], preferred_element_type=F32)
    x_intra = (jnp.dot(x[...], wroot[...], preferred_element_type=F32)
               + broot[...] + aggr / deg)
    gno = jax.nn.relu(x_intra + x_cross)
    y = gno * (1.0 + scale[...]) + shift[...]
    y = y * jax.nn.sigmoid(y)
    y_out[...] = jnp.dot(y + x[...], wdec[...], preferred_element_type=F32) + bdec[...]


def _combine_call(x_flow, tabff, tabmf, scale, shift, l, w):
    grid = N_FLOW // BN
    return pl.pallas_call(
        _combine_body,
        grid=(grid,),
        in_specs=[
            pl.BlockSpec((BN, EMB), lambda i: (i, 0)),
            pl.BlockSpec((2, BN, 128), lambda i: (0, i, 0)),
            pl.BlockSpec((2, BN, 128), lambda i: (0, i, 0)),
            pl.BlockSpec((EMB, EMB), lambda i: (0, 0)),
            pl.BlockSpec((1, EMB), lambda i: (0, 0)),
            pl.BlockSpec((EMB, EMB), lambda i: (0, 0)),
            pl.BlockSpec((1, EMB), lambda i: (0, 0)),
            pl.BlockSpec((1, EMB), lambda i: (0, 0)),
            pl.BlockSpec((EMB, 4), lambda i: (0, 0)),
            pl.BlockSpec((1, 4), lambda i: (0, 0)),
        ],
        out_specs=[pl.BlockSpec((BN, 4), lambda i: (i, 0))],
        out_shape=[jax.ShapeDtypeStruct((N_FLOW, 4), F32)],
    )(x_flow, tabff, tabmf, l["Wroot"], l["broot"][None, :], l["Wo"],
      scale, shift, w["dec_flow_W"], w["dec_flow_b"][None, :])[0]


def kernel(flow_x, memb_y, tau, edge_index_f2f, edge_attr_f2f,
           edge_index_m2f, edge_attr_m2f, weights):
    w = weights
    l = w["layers"][-1]

    half = TDIM // 2
    freqs = jnp.exp(-jnp.log(10000.0) * jnp.arange(half) / half)
    args = tau[:, None] * freqs[None, :]
    temb = jnp.concatenate([jnp.sin(args), jnp.cos(args)], axis=1)
    cf_row = temb @ w["enc_flow_W"][4:] + w["enc_flow_b"][None, :]
    cm_row = temb @ w["enc_memb_W1"][3:] + w["enc_memb_b1"][None, :]
    t = jax.nn.silu(tau.reshape(1, 1) @ l["tcW1"] + l["tcb1"][None, :])
    tc_row = t @ l["tcW2"] + l["tcb2"][None, :]
    scale, shift = tc_row[:, :EMB], tc_row[:, EMB:]

    src_ff = edge_index_f2f[0].astype(jnp.int32)
    dst_ff = edge_index_f2f[1].astype(jnp.int32)
    src_mf = edge_index_m2f[0].astype(jnp.int32)
    dst_mf = edge_index_m2f[1].astype(jnp.int32)
    zeros_ff = jnp.zeros((E_FF_P - E_FF,), jnp.int32)
    zeros_mf = jnp.zeros((E_MF_P - E_MF,), jnp.int32)
    src3 = jnp.concatenate([src_ff, zeros_ff]).reshape(NW, FF_CHUNKS, CHUNK)
    dstf3 = jnp.concatenate([dst_ff, zeros_ff + DUMMY]).reshape(NW, FF_CHUNKS, CHUNK)
    dst3 = jnp.concatenate([dst_mf, zeros_mf]).reshape(NW, MF_CHUNKS, CHUNK)
    srcm3 = jnp.concatenate([src_mf, zeros_mf]).reshape(NW, MF_CHUNKS, CHUNK)
    dstm3 = jnp.concatenate([dst_mf, zeros_mf + DUMMY]).reshape(NW, MF_CHUNKS, CHUNK)
    ea_ff_p = jnp.pad(edge_attr_f2f, ((0, E_FF_P - E_FF), (0, 0)))
    ea_mf_p = jnp.pad(edge_attr_m2f, ((0, E_MF_P - E_MF), (0, 0)))

    x_flow, tab_a, tab_b, ym_out = _enc_call(flow_x, memb_y, cf_row, cm_row, w)

    xs, qd, ys = _gather_call(tab_a, tab_b, src3, dst3, srcm3)

    vs, e3, gmax = _att_e_call(qd, ys, ea_mf_p, l)
    pmf = _att_a_call(e3, vs, gmax)
    pff = _gno_call(ea_ff_p, xs, l)

    tabff, tabmf = _scatter_call(pff, pmf, dstf3, dstm3, jnp.zeros((NT, 128), F32))

    y_flow_out = _combine_call(x_flow, tabff, tabmf, scale, shift, l, w)

    return (y_flow_out, ym_out)


# speedup vs baseline: 2.2695x; 1.0599x over previous
import functools

import jax
import jax.numpy as jnp
from jax import lax
from jax.experimental import pallas as pl
from jax.experimental.pallas import tpu as pltpu
from jax.experimental.pallas import tpu_sc as plsc

F32 = jnp.float32
N_FLOW = 10000
N_MEMB = 2000
E_FF = 160000
E_MF = 80000
EMB = 16
ATTN = 16
TDIM = 16

NW = 32
CHUNK = 128
FF_PER_W = 5120
MF_PER_W = 2560
E_FF_P = NW * FF_PER_W
E_MF_P = NW * MF_PER_W
FF_CHUNKS = FF_PER_W // CHUNK
MF_CHUNKS = MF_PER_W // CHUNK
NT = 10112
NT_PER_S = NT // 16
DUMMY = 10008

NB = 4
NBS = 2
BE = 2048
BF = 2048
BN = 2000

@functools.cache
def _sc_mesh():
    return plsc.VectorSubcoreMesh(core_axis_name="c", subcore_axis_name="s",
                                  num_cores=2, num_subcores=16)


def _enc_body(flow_x, memb_y, cf_row, cm_row, wf, w1m, w2m, b2, wq, wdm, bdm,
              x_flow, tab_a, tab_b, ym_out):
    xf = jnp.dot(flow_x[...], wf[...], preferred_element_type=F32) + cf_row[...]
    x_flow[...] = xf
    tab_a[:, :EMB] = xf
    tab_a[:, EMB:2 * EMB] = jnp.dot(xf, wq[...], preferred_element_type=F32)
    tab_a[:, 2 * EMB:] = jnp.zeros((N_FLOW, 128 - 2 * EMB), F32)
    ym1 = jax.nn.relu(jnp.dot(memb_y[...], w1m[...], preferred_element_type=F32) + cm_row[...])
    ym2 = jax.nn.relu(jnp.dot(ym1, w2m[...], preferred_element_type=F32) + b2[...])
    tab_b[:, :EMB] = ym2
    tab_b[:, EMB:] = jnp.zeros((N_MEMB, 128 - EMB), F32)
    ym_out[...] = jnp.dot(ym2, wdm[...], preferred_element_type=F32) + bdm[...]


def _enc_call(flow_x, memb_y, cf_row, cm_row, w):
    wf = w["enc_flow_W"][:4]
    w1m = w["enc_memb_W1"][:3]
    w2m = w["enc_memb_W2"]
    b2 = w["enc_memb_b2"][None, :]
    wq = w["layers"][-1]["Wq"]
    wdm = w["dec_memb_W"]
    bdm = w["dec_memb_b"][None, :]
    return pl.pallas_call(
        _enc_body,
        out_shape=(
            jax.ShapeDtypeStruct((N_FLOW, EMB), F32),
            jax.ShapeDtypeStruct((N_FLOW, 128), F32),
            jax.ShapeDtypeStruct((N_MEMB, 128), F32),
            jax.ShapeDtypeStruct((N_MEMB, 3), F32),
        ),
    )(flow_x, memb_y, cf_row, cm_row, wf, w1m, w2m, b2, wq, wdm, bdm)


def _gather_body(taba_hbm, tabb_hbm, src3, dst3, srcm3,
                 xs_out, qd_out, ys_out,
                 idx_ff, idx_mf1, idx_mf2, rows4, gsem, osem):
    cid = lax.axis_index("c")
    sid = lax.axis_index("s")
    wid = cid * 16 + sid

    pltpu.sync_copy(src3.at[wid], idx_ff)
    pltpu.sync_copy(dst3.at[wid], idx_mf1)
    pltpu.sync_copy(srcm3.at[wid], idx_mf2)

    def job(tab, idxbuf, out, base, ngroups):
        def grp(g, carry):
            gh = [pltpu.async_copy(tab.at[idxbuf.at[g * NB + b]], rows4.at[b],
                                   gsem.at[b]) for b in range(NB)]
            oh = []
            for b in range(NB):
                gh[b].wait()
                oh.append(pltpu.async_copy(
                    rows4.at[b],
                    out.at[pl.ds(base + (g * NB + b) * CHUNK, CHUNK)],
                    osem.at[b]))
            for h in oh:
                h.wait()
            return carry
        lax.fori_loop(0, ngroups, grp, 0)

    job(taba_hbm, idx_ff, xs_out, wid * FF_PER_W, FF_CHUNKS // NB)
    job(taba_hbm, idx_mf1, qd_out, wid * MF_PER_W, MF_CHUNKS // NB)
    job(tabb_hbm, idx_mf2, ys_out, wid * MF_PER_W, MF_CHUNKS // NB)


def _gather_call(tab_a, tab_b, src3, dst3, srcm3):
    call = pl.kernel(
        _gather_body,
        out_type=(
            jax.ShapeDtypeStruct((E_FF_P, 128), F32),
            jax.ShapeDtypeStruct((E_MF_P, 128), F32),
            jax.ShapeDtypeStruct((E_MF_P, 128), F32),
        ),
        mesh=_sc_mesh(),
        scratch_types=[
            pltpu.VMEM((FF_CHUNKS, CHUNK), jnp.int32),
            pltpu.VMEM((MF_CHUNKS, CHUNK), jnp.int32),
            pltpu.VMEM((MF_CHUNKS, CHUNK), jnp.int32),
            pltpu.VMEM((NB, CHUNK, 128), F32),
            pltpu.SemaphoreType.DMA((NB,)),
            pltpu.SemaphoreType.DMA((NB,)),
        ],
    )
    return call(tab_a, tab_b, src3, dst3, srcm3)


def _att_e_body(qd, ys, ea, wk, wv, we, vs_out, e_out, gmax_out):
    i = pl.program_id(0)
    ysb = ys[:, :EMB]
    qdb = qd[:, EMB:2 * EMB]
    ks = jnp.dot(ysb, wk[...], preferred_element_type=F32)
    vs = jnp.dot(ysb, wv[...], preferred_element_type=F32)
    vs_out[...] = vs
    ew = jnp.dot(ea[...], we[...], preferred_element_type=F32)
    e = jnp.sum(qdb * ks, axis=1) * (1.0 / 4.0) + ew[:, 0]
    e_out[0, 0, :] = e
    bmax = jnp.max(e)

    @pl.when(i == 0)
    def _():
        gmax_out[...] = jnp.full((1, 1), bmax, F32)

    @pl.when(i > 0)
    def _():
        gmax_out[...] = jnp.maximum(gmax_out[...], bmax)


def _att_e_call(qd, ys, ea_mf_p, l):
    grid = E_MF_P // BE
    return pl.pallas_call(
        _att_e_body,
        grid=(grid,),
        in_specs=[
            pl.BlockSpec((BE, 128), lambda i: (i, 0)),
            pl.BlockSpec((BE, 128), lambda i: (i, 0)),
            pl.BlockSpec((BE, 4), lambda i: (i, 0)),
            pl.BlockSpec((ATTN, ATTN), lambda i: (0, 0)),
            pl.BlockSpec((EMB, EMB), lambda i: (0, 0)),
            pl.BlockSpec((4, 1), lambda i: (0, 0)),
        ],
        out_specs=[
            pl.BlockSpec((BE, EMB), lambda i: (i, 0)),
            pl.BlockSpec((1, 1, BE), lambda i: (i, 0, 0)),
            pl.BlockSpec((1, 1), lambda i: (0, 0)),
        ],
        out_shape=[
            jax.ShapeDtypeStruct((E_MF_P, EMB), F32),
            jax.ShapeDtypeStruct((grid, 1, BE), F32),
            jax.ShapeDtypeStruct((1, 1), F32),
        ],
    )(qd, ys, ea_mf_p, l["Wk"], l["Wv"], l["We"])


def _att_a_body(e3, vs, gmax, p_out):
    a = jnp.exp(e3[0, 0, :] - gmax[0, 0])
    p_out[:, :EMB] = a[:, None] * vs[...]
    p_out[:, EMB:EMB + 1] = a[:, None]
    p_out[:, EMB + 1:] = jnp.zeros((BE, 128 - EMB - 1), F32)


def _att_a_call(e3, vs, gmax):
    grid = E_MF_P // BE
    return pl.pallas_call(
        _att_a_body,
        grid=(grid,),
        in_specs=[
            pl.BlockSpec((1, 1, BE), lambda i: (i, 0, 0)),
            pl.BlockSpec((BE, EMB), lambda i: (i, 0)),
            pl.BlockSpec((1, 1), lambda i: (0, 0)),
        ],
        out_specs=[pl.BlockSpec((BE, 128), lambda i: (i, 0))],
        out_shape=[jax.ShapeDtypeStruct((E_MF_P, 128), F32)],
    )(e3, vs, gmax)[0]


def _gno_body(ea, xs, w1, b1, w2, b2, w3, b3, p_out):
    h = jax.nn.relu(jnp.dot(ea[...], w1[...], preferred_element_type=F32) + b1[...])
    h = jax.nn.relu(jnp.dot(h, w2[...], preferred_element_type=F32) + b2[...])
    k = jnp.dot(h, w3[...], preferred_element_type=F32) + b3[...]
    xt = jnp.concatenate([xs[:, :EMB]] * EMB, axis=1)
    grp = lax.broadcasted_iota(jnp.int32, (EMB * EMB, EMB), 0) // EMB
    col = lax.broadcasted_iota(jnp.int32, (EMB * EMB, EMB), 1)
    s = (grp == col).astype(F32)
    m = jnp.dot(k * xt, s, preferred_element_type=F32)
    p_out[:, :EMB] = m
    p_out[:, EMB:EMB + 1] = jnp.ones((BF, 1), F32)
    p_out[:, EMB + 1:] = jnp.zeros((BF, 128 - EMB - 1), F32)


def _gno_call(ea_ff_p, xs, l):
    grid = E_FF_P // BF
    return pl.pallas_call(
        _gno_body,
        grid=(grid,),
        in_specs=[
            pl.BlockSpec((BF, 4), lambda i: (i, 0)),
            pl.BlockSpec((BF, 128), lambda i: (i, 0)),
            pl.BlockSpec((4, 128), lambda i: (0, 0)),
            pl.BlockSpec((1, 128), lambda i: (0, 0)),
            pl.BlockSpec((128, 128), lambda i: (0, 0)),
            pl.BlockSpec((1, 128), lambda i: (0, 0)),
            pl.BlockSpec((128, 256), lambda i: (0, 0)),
            pl.BlockSpec((1, 256), lambda i: (0, 0)),
        ],
        out_specs=[pl.BlockSpec((BF, 128), lambda i: (i, 0))],
        out_shape=[jax.ShapeDtypeStruct((E_FF_P, 128), F32)],
    )(ea_ff_p, xs, l["kW1"], l["kb1"][None, :], l["kW2"], l["kb2"][None, :],
      l["kW3"], l["kb3"][None, :])[0]


def _scatter_body(pff_hbm, pmf_hbm, dstf3, dstm3, zeros_hbm,
                  tabff_out, tabmf_out,
                  idx_ff, idx_mf, pbuf4, tab_sh, rsem, ssem):
    cid = lax.axis_index("c")
    sid = lax.axis_index("s")
    wid = cid * 16 + sid
    zrow = pl.ds(sid * NT_PER_S, NT_PER_S)

    pltpu.sync_copy(dstf3.at[wid], idx_ff)
    pltpu.sync_copy(dstm3.at[wid], idx_mf)

    def job(phbm, idxbuf, base, ngroups):
        def grp(g, carry):
            rh = [pltpu.async_copy(
                phbm.at[pl.ds(base + (g * NBS + b) * CHUNK, CHUNK)],
                pbuf4.at[b], rsem.at[b]) for b in range(NBS)]
            sh = []
            for b in range(NBS):
                rh[b].wait()
                sh.append(pltpu.async_copy(pbuf4.at[b],
                                           tab_sh.at[idxbuf.at[g * NBS + b]],
                                           ssem.at[b], add=True))
            for h in sh:
                h.wait()
            return carry
        lax.fori_loop(0, ngroups, grp, 0)

    pltpu.sync_copy(zeros_hbm.at[zrow], tab_sh.at[zrow])
    plsc.subcore_barrier()
    job(pff_hbm, idx_ff, wid * FF_PER_W, FF_CHUNKS // NBS)
    plsc.subcore_barrier()
    pltpu.sync_copy(tab_sh.at[zrow], tabff_out.at[cid, zrow])
    plsc.subcore_barrier()

    pltpu.sync_copy(zeros_hbm.at[zrow], tab_sh.at[zrow])
    plsc.subcore_barrier()
    job(pmf_hbm, idx_mf, wid * MF_PER_W, MF_CHUNKS // NBS)
    plsc.subcore_barrier()
    pltpu.sync_copy(tab_sh.at[zrow], tabmf_out.at[cid, zrow])


def _scatter_call(pff, pmf, dstf3, dstm3, zeros_hbm):
    call = pl.kernel(
        _scatter_body,
        out_type=(
            jax.ShapeDtypeStruct((2, NT, 128), F32),
            jax.ShapeDtypeStruct((2, NT, 128), F32),
        ),
        mesh=_sc_mesh(),
        scratch_types=[
            pltpu.VMEM((FF_CHUNKS, CHUNK), jnp.int32),
            pltpu.VMEM((MF_CHUNKS, CHUNK), jnp.int32),
            pltpu.VMEM((NBS, CHUNK, 128), F32),
            pltpu.VMEM_SHARED((NT, 128), F32),
            pltpu.SemaphoreType.DMA((NBS,)),
            pltpu.SemaphoreType.DMA((NBS,)),
        ],
    )
    return call(pff, pmf, dstf3, dstm3, zeros_hbm)


def _combine_body(x, tabff, tabmf, wroot, broot, wo, scale, shift, wdec, bdec, y_out):
    tf = tabff[0] + tabff[1]
    tm = tabmf[0] + tabmf[1]
    aggr = tf[:, :EMB]
    deg = jnp.maximum(tf[:, EMB:EMB + 1], 1.0)
    msg = tm[:, :EMB]
    denom = jnp.maximum(tm[:, EMB:EMB + 1], 1e-30)
    x_cross = jnp.dot(msg / denom, wo[...], preferred_element_type=F32)
    x_intra = (jnp.dot(x[...], wroot[...], preferred_element_type=F32)
               + broot[...] + aggr / deg)
    gno = jax.nn.relu(x_intra + x_cross)
    y = gno * (1.0 + scale[...]) + shift[...]
    y = y * jax.nn.sigmoid(y)
    y_out[...] = jnp.dot(y + x[...], wdec[...], preferred_element_type=F32) + bdec[...]


def _combine_call(x_flow, tabff, tabmf, scale, shift, l, w):
    grid = N_FLOW // BN
    return pl.pallas_call(
        _combine_body,
        grid=(grid,),
        in_specs=[
            pl.BlockSpec((BN, EMB), lambda i: (i, 0)),
            pl.BlockSpec((2, BN, 128), lambda i: (0, i, 0)),
            pl.BlockSpec((2, BN, 128), lambda i: (0, i, 0)),
            pl.BlockSpec((EMB, EMB), lambda i: (0, 0)),
            pl.BlockSpec((1, EMB), lambda i: (0, 0)),
            pl.BlockSpec((EMB, EMB), lambda i: (0, 0)),
            pl.BlockSpec((1, EMB), lambda i: (0, 0)),
            pl.BlockSpec((1, EMB), lambda i: (0, 0)),
            pl.BlockSpec((EMB, 4), lambda i: (0, 0)),
            pl.BlockSpec((1, 4), lambda i: (0, 0)),
        ],
        out_specs=[pl.BlockSpec((BN, 4), lambda i: (i, 0))],
        out_shape=[jax.ShapeDtypeStruct((N_FLOW, 4), F32)],
    )(x_flow, tabff, tabmf, l["Wroot"], l["broot"][None, :], l["Wo"],
      scale, shift, w["dec_flow_W"], w["dec_flow_b"][None, :])[0]


def kernel(flow_x, memb_y, tau, edge_index_f2f, edge_attr_f2f,
           edge_index_m2f, edge_attr_m2f, weights):
    w = weights
    l = w["layers"][-1]

    half = TDIM // 2
    freqs = jnp.exp(-jnp.log(10000.0) * jnp.arange(half) / half)
    args = tau[:, None] * freqs[None, :]
    temb = jnp.concatenate([jnp.sin(args), jnp.cos(args)], axis=1)
    cf_row = temb @ w["enc_flow_W"][4:] + w["enc_flow_b"][None, :]
    cm_row = temb @ w["enc_memb_W1"][3:] + w["enc_memb_b1"][None, :]
    t = jax.nn.silu(tau.reshape(1, 1) @ l["tcW1"] + l["tcb1"][None, :])
    tc_row = t @ l["tcW2"] + l["tcb2"][None, :]
    scale, shift = tc_row[:, :EMB], tc_row[:, EMB:]

    src_ff = edge_index_f2f[0].astype(jnp.int32)
    dst_ff = edge_index_f2f[1].astype(jnp.int32)
    src_mf = edge_index_m2f[0].astype(jnp.int32)
    dst_mf = edge_index_m2f[1].astype(jnp.int32)
    zeros_ff = jnp.zeros((E_FF_P - E_FF,), jnp.int32)
    zeros_mf = jnp.zeros((E_MF_P - E_MF,), jnp.int32)
    src3 = jnp.concatenate([src_ff, zeros_ff]).reshape(NW, FF_CHUNKS, CHUNK)
    dstf3 = jnp.concatenate([dst_ff, zeros_ff + DUMMY]).reshape(NW, FF_CHUNKS, CHUNK)
    dst3 = jnp.concatenate([dst_mf, zeros_mf]).reshape(NW, MF_CHUNKS, CHUNK)
    srcm3 = jnp.concatenate([src_mf, zeros_mf]).reshape(NW, MF_CHUNKS, CHUNK)
    dstm3 = jnp.concatenate([dst_mf, zeros_mf + DUMMY]).reshape(NW, MF_CHUNKS, CHUNK)
    ea_ff_p = jnp.pad(edge_attr_f2f, ((0, E_FF_P - E_FF), (0, 0)))
    ea_mf_p = jnp.pad(edge_attr_m2f, ((0, E_MF_P - E_MF), (0, 0)))

    x_flow, tab_a, tab_b, ym_out = _enc_call(flow_x, memb_y, cf_row, cm_row, w)

    xs, qd, ys = _gather_call(tab_a, tab_b, src3, dst3, srcm3)

    vs, e3, gmax = _att_e_call(qd, ys, ea_mf_p, l)
    pmf = _att_a_call(e3, vs, gmax)
    pff = _gno_call(ea_ff_p, xs, l)

    tabff, tabmf = _scatter_call(pff, pmf, dstf3, dstm3, jnp.zeros((NT, 128), F32))

    y_flow_out = _combine_call(x_flow, tabff, tabmf, scale, shift, l, w)

    return (y_flow_out, ym_out)


# speedup vs baseline: 3.4640x; 1.5263x over previous
import functools

import jax
import jax.numpy as jnp
from jax import lax
from jax.experimental import pallas as pl
from jax.experimental.pallas import tpu as pltpu
from jax.experimental.pallas import tpu_sc as plsc

F32 = jnp.float32
N_FLOW = 10000
N_MEMB = 2000
E_FF = 160000
E_MF = 80000
EMB = 16
ATTN = 16
TDIM = 16

NW = 32
CHUNK = 128
FF_PER_W = 5120
MF_PER_W = 2560
E_FF_P = NW * FF_PER_W
E_MF_P = NW * MF_PER_W
FF_CHUNKS = FF_PER_W // CHUNK
MF_CHUNKS = MF_PER_W // CHUNK
NT = 10112
NT_PER_S = NT // 16
DUMMY = 10008

NB = 4
BE = 2048
BF = 2048
BN = 2000

_SC_PARAMS = pltpu.CompilerParams(use_tc_tiling_on_sc=False)


@functools.cache
def _sc_mesh():
    return plsc.VectorSubcoreMesh(core_axis_name="c", subcore_axis_name="s",
                                  num_cores=2, num_subcores=16)


def _enc_body(flow_x, memb_y, cf_row, cm_row, wf, w1m, w2m, b2, wq, wdm, bdm,
              x_flow, q_full, y_memb, ym_out):
    xf = jnp.dot(flow_x[...], wf[...], preferred_element_type=F32) + cf_row[...]
    x_flow[...] = xf
    q_full[...] = jnp.dot(xf, wq[...], preferred_element_type=F32)
    ym1 = jax.nn.relu(jnp.dot(memb_y[...], w1m[...], preferred_element_type=F32) + cm_row[...])
    ym2 = jax.nn.relu(jnp.dot(ym1, w2m[...], preferred_element_type=F32) + b2[...])
    y_memb[...] = ym2
    ym_out[...] = jnp.dot(ym2, wdm[...], preferred_element_type=F32) + bdm[...]


def _enc_call(flow_x, memb_y, cf_row, cm_row, w):
    wf = w["enc_flow_W"][:4]
    w1m = w["enc_memb_W1"][:3]
    w2m = w["enc_memb_W2"]
    b2 = w["enc_memb_b2"][None, :]
    wq = w["layers"][-1]["Wq"]
    wdm = w["dec_memb_W"]
    bdm = w["dec_memb_b"][None, :]
    return pl.pallas_call(
        _enc_body,
        out_shape=(
            jax.ShapeDtypeStruct((N_FLOW, EMB), F32),
            jax.ShapeDtypeStruct((N_FLOW, ATTN), F32),
            jax.ShapeDtypeStruct((N_MEMB, EMB), F32),
            jax.ShapeDtypeStruct((N_MEMB, 3), F32),
        ),
    )(flow_x, memb_y, cf_row, cm_row, wf, w1m, w2m, b2, wq, wdm, bdm)


def _gather_body(tabx_hbm, tabq_hbm, tabb_hbm, src3, dst3, srcm3,
                 xs_out, qd_out, ys_out,
                 idx_ff, idx_mf1, idx_mf2, rows4, gsem, osem):
    cid = lax.axis_index("c")
    sid = lax.axis_index("s")
    wid = cid * 16 + sid

    pltpu.sync_copy(src3.at[wid], idx_ff)
    pltpu.sync_copy(dst3.at[wid], idx_mf1)
    pltpu.sync_copy(srcm3.at[wid], idx_mf2)

    def job(tab, idxbuf, out, base, ngroups):
        def grp(g, carry):
            gh = [pltpu.async_copy(tab.at[idxbuf.at[g * NB + b]], rows4.at[b],
                                   gsem.at[b]) for b in range(NB)]
            oh = []
            for b in range(NB):
                gh[b].wait()
                oh.append(pltpu.async_copy(
                    rows4.at[b],
                    out.at[pl.ds(base + (g * NB + b) * CHUNK, CHUNK)],
                    osem.at[b]))
            for h in oh:
                h.wait()
            return carry
        lax.fori_loop(0, ngroups, grp, 0)

    job(tabx_hbm, idx_ff, xs_out, wid * FF_PER_W, FF_CHUNKS // NB)
    job(tabq_hbm, idx_mf1, qd_out, wid * MF_PER_W, MF_CHUNKS // NB)
    job(tabb_hbm, idx_mf2, ys_out, wid * MF_PER_W, MF_CHUNKS // NB)


def _gather_call(tabx, tabq, tabb, src3, dst3, srcm3):
    call = pl.kernel(
        _gather_body,
        out_type=(
            jax.ShapeDtypeStruct((E_FF_P, EMB), F32),
            jax.ShapeDtypeStruct((E_MF_P, EMB), F32),
            jax.ShapeDtypeStruct((E_MF_P, EMB), F32),
        ),
        mesh=_sc_mesh(),
        compiler_params=_SC_PARAMS,
        scratch_types=[
            pltpu.VMEM((FF_CHUNKS, CHUNK), jnp.int32),
            pltpu.VMEM((MF_CHUNKS, CHUNK), jnp.int32),
            pltpu.VMEM((MF_CHUNKS, CHUNK), jnp.int32),
            pltpu.VMEM((NB, CHUNK, EMB), F32),
            pltpu.SemaphoreType.DMA((NB,)),
            pltpu.SemaphoreType.DMA((NB,)),
        ],
    )
    return call(tabx, tabq, tabb, src3, dst3, srcm3)


def _att_e_body(qd, ys, ea, wk, wv, we, vs_out, e_out, gmax_out):
    i = pl.program_id(0)
    ks = jnp.dot(ys[...], wk[...], preferred_element_type=F32)
    vs = jnp.dot(ys[...], wv[...], preferred_element_type=F32)
    vs_out[...] = vs
    ew = jnp.dot(ea[...], we[...], preferred_element_type=F32)
    e = jnp.sum(qd[...] * ks, axis=1) * (1.0 / 4.0) + ew[:, 0]
    e_out[0, 0, :] = e
    bmax = jnp.max(e)

    @pl.when(i == 0)
    def _():
        gmax_out[...] = jnp.full((1, 1), bmax, F32)

    @pl.when(i > 0)
    def _():
        gmax_out[...] = jnp.maximum(gmax_out[...], bmax)


def _att_e_call(qd, ys, ea_mf_p, l):
    grid = E_MF_P // BE
    return pl.pallas_call(
        _att_e_body,
        grid=(grid,),
        in_specs=[
            pl.BlockSpec((BE, ATTN), lambda i: (i, 0)),
            pl.BlockSpec((BE, EMB), lambda i: (i, 0)),
            pl.BlockSpec((BE, 4), lambda i: (i, 0)),
            pl.BlockSpec((ATTN, ATTN), lambda i: (0, 0)),
            pl.BlockSpec((EMB, EMB), lambda i: (0, 0)),
            pl.BlockSpec((4, 1), lambda i: (0, 0)),
        ],
        out_specs=[
            pl.BlockSpec((BE, EMB), lambda i: (i, 0)),
            pl.BlockSpec((1, 1, BE), lambda i: (i, 0, 0)),
            pl.BlockSpec((1, 1), lambda i: (0, 0)),
        ],
        out_shape=[
            jax.ShapeDtypeStruct((E_MF_P, EMB), F32),
            jax.ShapeDtypeStruct((grid, 1, BE), F32),
            jax.ShapeDtypeStruct((1, 1), F32),
        ],
    )(qd, ys, ea_mf_p, l["Wk"], l["Wv"], l["We"])


def _att_a_body(e3, vs, gmax, p_out):
    a = jnp.exp(e3[0, 0, :] - gmax[0, 0])
    p_out[:, :EMB] = a[:, None] * vs[...]
    p_out[:, EMB:EMB + 1] = a[:, None]
    p_out[:, EMB + 1:] = jnp.zeros((BE, 32 - EMB - 1), F32)


def _att_a_call(e3, vs, gmax):
    grid = E_MF_P // BE
    return pl.pallas_call(
        _att_a_body,
        grid=(grid,),
        in_specs=[
            pl.BlockSpec((1, 1, BE), lambda i: (i, 0, 0)),
            pl.BlockSpec((BE, EMB), lambda i: (i, 0)),
            pl.BlockSpec((1, 1), lambda i: (0, 0)),
        ],
        out_specs=[pl.BlockSpec((BE, 32), lambda i: (i, 0))],
        out_shape=[jax.ShapeDtypeStruct((E_MF_P, 32), F32)],
    )(e3, vs, gmax)[0]


def _gno_body(ea, xs, w1, b1, w2, b2, w3, b3, p_out):
    h = jax.nn.relu(jnp.dot(ea[...], w1[...], preferred_element_type=F32) + b1[...])
    h = jax.nn.relu(jnp.dot(h, w2[...], preferred_element_type=F32) + b2[...])
    k = jnp.dot(h, w3[...], preferred_element_type=F32) + b3[...]
    xt = jnp.concatenate([xs[...]] * EMB, axis=1)
    grp = lax.broadcasted_iota(jnp.int32, (EMB * EMB, EMB), 0) // EMB
    col = lax.broadcasted_iota(jnp.int32, (EMB * EMB, EMB), 1)
    s = (grp == col).astype(F32)
    m = jnp.dot(k * xt, s, preferred_element_type=F32)
    p_out[:, :EMB] = m
    p_out[:, EMB:EMB + 1] = jnp.ones((BF, 1), F32)
    p_out[:, EMB + 1:] = jnp.zeros((BF, 32 - EMB - 1), F32)


def _gno_call(ea_ff_p, xs, l):
    grid = E_FF_P // BF
    return pl.pallas_call(
        _gno_body,
        grid=(grid,),
        in_specs=[
            pl.BlockSpec((BF, 4), lambda i: (i, 0)),
            pl.BlockSpec((BF, EMB), lambda i: (i, 0)),
            pl.BlockSpec((4, 128), lambda i: (0, 0)),
            pl.BlockSpec((1, 128), lambda i: (0, 0)),
            pl.BlockSpec((128, 128), lambda i: (0, 0)),
            pl.BlockSpec((1, 128), lambda i: (0, 0)),
            pl.BlockSpec((128, 256), lambda i: (0, 0)),
            pl.BlockSpec((1, 256), lambda i: (0, 0)),
        ],
        out_specs=[pl.BlockSpec((BF, 32), lambda i: (i, 0))],
        out_shape=[jax.ShapeDtypeStruct((E_FF_P, 32), F32)],
    )(ea_ff_p, xs, l["kW1"], l["kb1"][None, :], l["kW2"], l["kb2"][None, :],
      l["kW3"], l["kb3"][None, :])[0]


def _scatter_body(pff_hbm, pmf_hbm, dstf3, dstm3,
                  tabff_out, tabmf_out,
                  idx_ff, idx_mf, pbuf4, zb, tabff_sh, tabmf_sh, rsem, ssem):
    cid = lax.axis_index("c")
    sid = lax.axis_index("s")
    wid = cid * 16 + sid
    zrow = pl.ds(sid * NT_PER_S, NT_PER_S)

    pltpu.sync_copy(dstf3.at[wid], idx_ff)
    pltpu.sync_copy(dstm3.at[wid], idx_mf)

    z16 = jnp.zeros((EMB,), F32)

    def zfill(r, carry):
        zb[r, :EMB] = z16
        zb[r, EMB:2 * EMB] = z16
        return carry

    lax.fori_loop(0, NT_PER_S, zfill, 0)
    pltpu.sync_copy(zb, tabff_sh.at[zrow])
    pltpu.sync_copy(zb, tabmf_sh.at[zrow])
    plsc.subcore_barrier()

    def job(phbm, idxbuf, tab_sh, base, ngroups):
        def grp(g, carry):
            rh = [pltpu.async_copy(
                phbm.at[pl.ds(base + (g * NB + b) * CHUNK, CHUNK)],
                pbuf4.at[b], rsem.at[b]) for b in range(NB)]
            sh = []
            for b in range(NB):
                rh[b].wait()
                sh.append(pltpu.async_copy(pbuf4.at[b],
                                           tab_sh.at[idxbuf.at[g * NB + b]],
                                           ssem.at[b], add=True))
            for h in sh:
                h.wait()
            return carry
        lax.fori_loop(0, ngroups, grp, 0)

    job(pff_hbm, idx_ff, tabff_sh, wid * FF_PER_W, FF_CHUNKS // NB)
    job(pmf_hbm, idx_mf, tabmf_sh, wid * MF_PER_W, MF_CHUNKS // NB)

    plsc.subcore_barrier()
    pltpu.sync_copy(tabff_sh.at[zrow], tabff_out.at[cid, zrow])
    pltpu.sync_copy(tabmf_sh.at[zrow], tabmf_out.at[cid, zrow])


def _scatter_call(pff, pmf, dstf3, dstm3):
    call = pl.kernel(
        _scatter_body,
        out_type=(
            jax.ShapeDtypeStruct((2, NT, 32), F32),
            jax.ShapeDtypeStruct((2, NT, 32), F32),
        ),
        mesh=_sc_mesh(),
        compiler_params=_SC_PARAMS,
        scratch_types=[
            pltpu.VMEM((FF_CHUNKS, CHUNK), jnp.int32),
            pltpu.VMEM((MF_CHUNKS, CHUNK), jnp.int32),
            pltpu.VMEM((NB, CHUNK, 32), F32),
            pltpu.VMEM((NT_PER_S, 32), F32),
            pltpu.VMEM_SHARED((NT, 32), F32),
            pltpu.VMEM_SHARED((NT, 32), F32),
            pltpu.SemaphoreType.DMA((NB,)),
            pltpu.SemaphoreType.DMA((NB,)),
        ],
    )
    return call(pff, pmf, dstf3, dstm3)


def _combine_body(x, tabff, tabmf, wroot, broot, wo, scale, shift, wdec, bdec, y_out):
    tf = tabff[0] + tabff[1]
    tm = tabmf[0] + tabmf[1]
    aggr = tf[:, :EMB]
    deg = jnp.maximum(tf[:, EMB:EMB + 1], 1.0)
    msg = tm[:, :EMB]
    denom = jnp.maximum(tm[:, EMB:EMB + 1], 1e-30)
    x_cross = jnp.dot(msg / denom, wo[...], preferred_element_type=F32)
    x_intra = (jnp.dot(x[...], wroot[...], preferred_element_type=F32)
               + broot[...] + aggr / deg)
    gno = jax.nn.relu(x_intra + x_cross)
    y = gno * (1.0 + scale[...]) + shift[...]
    y = y * jax.nn.sigmoid(y)
    y_out[...] = jnp.dot(y + x[...], wdec[...], preferred_element_type=F32) + bdec[...]


def _combine_call(x_flow, tabff, tabmf, scale, shift, l, w):
    grid = N_FLOW // BN
    return pl.pallas_call(
        _combine_body,
        grid=(grid,),
        in_specs=[
            pl.BlockSpec((BN, EMB), lambda i: (i, 0)),
            pl.BlockSpec((2, BN, 32), lambda i: (0, i, 0)),
            pl.BlockSpec((2, BN, 32), lambda i: (0, i, 0)),
            pl.BlockSpec((EMB, EMB), lambda i: (0, 0)),
            pl.BlockSpec((1, EMB), lambda i: (0, 0)),
            pl.BlockSpec((EMB, EMB), lambda i: (0, 0)),
            pl.BlockSpec((1, EMB), lambda i: (0, 0)),
            pl.BlockSpec((1, EMB), lambda i: (0, 0)),
            pl.BlockSpec((EMB, 4), lambda i: (0, 0)),
            pl.BlockSpec((1, 4), lambda i: (0, 0)),
        ],
        out_specs=[pl.BlockSpec((BN, 4), lambda i: (i, 0))],
        out_shape=[jax.ShapeDtypeStruct((N_FLOW, 4), F32)],
    )(x_flow, tabff, tabmf, l["Wroot"], l["broot"][None, :], l["Wo"],
      scale, shift, w["dec_flow_W"], w["dec_flow_b"][None, :])[0]


def kernel(flow_x, memb_y, tau, edge_index_f2f, edge_attr_f2f,
           edge_index_m2f, edge_attr_m2f, weights):
    w = weights
    l = w["layers"][-1]

    half = TDIM // 2
    freqs = jnp.exp(-jnp.log(10000.0) * jnp.arange(half) / half)
    args = tau[:, None] * freqs[None, :]
    temb = jnp.concatenate([jnp.sin(args), jnp.cos(args)], axis=1)
    cf_row = temb @ w["enc_flow_W"][4:] + w["enc_flow_b"][None, :]
    cm_row = temb @ w["enc_memb_W1"][3:] + w["enc_memb_b1"][None, :]
    t = jax.nn.silu(tau.reshape(1, 1) @ l["tcW1"] + l["tcb1"][None, :])
    tc_row = t @ l["tcW2"] + l["tcb2"][None, :]
    scale, shift = tc_row[:, :EMB], tc_row[:, EMB:]

    src_ff = edge_index_f2f[0].astype(jnp.int32)
    dst_ff = edge_index_f2f[1].astype(jnp.int32)
    src_mf = edge_index_m2f[0].astype(jnp.int32)
    dst_mf = edge_index_m2f[1].astype(jnp.int32)
    zeros_ff = jnp.zeros((E_FF_P - E_FF,), jnp.int32)
    zeros_mf = jnp.zeros((E_MF_P - E_MF,), jnp.int32)
    src3 = jnp.concatenate([src_ff, zeros_ff]).reshape(NW, FF_CHUNKS, CHUNK)
    dstf3 = jnp.concatenate([dst_ff, zeros_ff + DUMMY]).reshape(NW, FF_CHUNKS, CHUNK)
    dst3 = jnp.concatenate([dst_mf, zeros_mf]).reshape(NW, MF_CHUNKS, CHUNK)
    srcm3 = jnp.concatenate([src_mf, zeros_mf]).reshape(NW, MF_CHUNKS, CHUNK)
    dstm3 = jnp.concatenate([dst_mf, zeros_mf + DUMMY]).reshape(NW, MF_CHUNKS, CHUNK)
    ea_ff_p = jnp.pad(edge_attr_f2f, ((0, E_FF_P - E_FF), (0, 0)))
    ea_mf_p = jnp.pad(edge_attr_m2f, ((0, E_MF_P - E_MF), (0, 0)))

    x_flow, q_full, y_memb, ym_out = _enc_call(flow_x, memb_y, cf_row, cm_row, w)

    xs, qd, ys = _gather_call(x_flow, q_full, y_memb, src3, dst3, srcm3)

    vs, e3, gmax = _att_e_call(qd, ys, ea_mf_p, l)
    pmf = _att_a_call(e3, vs, gmax)
    pff = _gno_call(ea_ff_p, xs, l)

    tabff, tabmf = _scatter_call(pff, pmf, dstf3, dstm3)

    y_flow_out = _combine_call(x_flow, tabff, tabmf, scale, shift, l, w)

    return (y_flow_out, ym_out)
